# Initial kernel scaffold; baseline (speedup 1.0000x reference)
import jax
import jax.numpy as jnp
from jax.experimental import pallas as pl


def kernel(x, W1, b1, W2, b2, W3, b3):
    raise NotImplementedError("write your pallas kernel here")


# speedup vs baseline: 1.2055x; 1.2055x over previous
import functools

import jax
import jax.numpy as jnp
from jax.experimental import pallas as pl

B = 16384
D = 4096
H1 = 256
H2 = 128
E = 64
TOP_K = 8

BLK = 1024


def _gating_body(x_ref, w1_ref, b1_ref, w2_ref, b2_ref, w3_ref, b3_ref,
                 scores_ref, idx_ref, topv_ref):
    x = x_ref[...]
    h = jnp.dot(x, w1_ref[...], preferred_element_type=jnp.float32)
    h = jnp.maximum(h + b1_ref[...], 0.0)
    h = jnp.dot(h, w2_ref[...], preferred_element_type=jnp.float32)
    h = jnp.maximum(h + b2_ref[...], 0.0)
    logits = jnp.dot(h, w3_ref[...], preferred_element_type=jnp.float32)
    logits = logits + b3_ref[...]

    m = jnp.max(logits, axis=1, keepdims=True)
    e = jnp.exp(logits - m)
    s = e / jnp.sum(e, axis=1, keepdims=True)
    scores_ref[...] = s

    col = jax.lax.broadcasted_iota(jnp.int32, s.shape, 1)
    work = s
    vals = []
    idxs = []
    for _ in range(TOP_K):
        mx = jnp.max(work, axis=1, keepdims=True)
        ind = jnp.min(jnp.where(work == mx, col, E), axis=1, keepdims=True)
        vals.append(mx)
        idxs.append(ind)
        work = jnp.where(col == ind, -1.0, work)
    v = jnp.concatenate(vals, axis=1)
    i = jnp.concatenate(idxs, axis=1)
    topv_ref[...] = v / jnp.sum(v, axis=1, keepdims=True)
    idx_ref[...] = i


@jax.jit
def _gating(x, w1t, b1, w2t, b2, w3t, b3):
    grid = (B // BLK,)
    out = pl.pallas_call(
        _gating_body,
        grid=grid,
        in_specs=[
            pl.BlockSpec((BLK, D), lambda i: (i, 0)),
            pl.BlockSpec((D, H1), lambda i: (0, 0)),
            pl.BlockSpec((1, H1), lambda i: (0, 0)),
            pl.BlockSpec((H1, H2), lambda i: (0, 0)),
            pl.BlockSpec((1, H2), lambda i: (0, 0)),
            pl.BlockSpec((H2, E), lambda i: (0, 0)),
            pl.BlockSpec((1, E), lambda i: (0, 0)),
        ],
        out_specs=[
            pl.BlockSpec((BLK, E), lambda i: (i, 0)),
            pl.BlockSpec((BLK, TOP_K), lambda i: (i, 0)),
            pl.BlockSpec((BLK, TOP_K), lambda i: (i, 0)),
        ],
        out_shape=[
            jax.ShapeDtypeStruct((B, E), jnp.float32),
            jax.ShapeDtypeStruct((B, TOP_K), jnp.int32),
            jax.ShapeDtypeStruct((B, TOP_K), jnp.float32),
        ],
    )(x, w1t, b1, w2t, b2, w3t, b3)
    return out


def kernel(x, W1, b1, W2, b2, W3, b3):
    gate_scores, top_k_indices, top_k_scores = _gating(
        x,
        W1.T, b1.reshape(1, H1),
        W2.T, b2.reshape(1, H2),
        W3.T, b3.reshape(1, E),
    )
    return (gate_scores, top_k_indices, top_k_scores)


# speedup vs baseline: 1.3187x; 1.0939x over previous
import functools

import jax
import jax.numpy as jnp
from jax.experimental import pallas as pl

B = 16384
D = 4096
H1 = 256
H2 = 128
E = 64
TOP_K = 8

BLK = 1024


def _gating_body(x_ref, w1_ref, b1_ref, w2_ref, b2_ref, w3_ref, b3_ref,
                 scores_ref, idx_ref, topv_ref):
    x = x_ref[...]
    h = jnp.dot(x, w1_ref[...], preferred_element_type=jnp.float32)
    h = jnp.maximum(h + b1_ref[...], 0.0)
    h = jnp.dot(h, w2_ref[...], preferred_element_type=jnp.float32)
    h = jnp.maximum(h + b2_ref[...], 0.0)
    logits = jnp.dot(h, w3_ref[...], preferred_element_type=jnp.float32)
    logits = logits + b3_ref[...]

    m = jnp.max(logits, axis=1, keepdims=True)
    e = jnp.exp(logits - m)
    s = e / jnp.sum(e, axis=1, keepdims=True)
    scores_ref[...] = s

    col = jax.lax.broadcasted_iota(jnp.int32, s.shape, 1)
    work = s
    vals = []
    idxs = []
    for _ in range(TOP_K):
        mx = jnp.max(work, axis=1, keepdims=True)
        ind = jnp.argmax(work, axis=1, keepdims=True)
        vals.append(mx)
        idxs.append(ind)
        work = jnp.where(col == ind, -1.0, work)
    v = jnp.concatenate(vals, axis=1)
    i = jnp.concatenate(idxs, axis=1)
    topv_ref[...] = v / jnp.sum(v, axis=1, keepdims=True)
    idx_ref[...] = i


@jax.jit
def _gating(x, w1t, b1, w2t, b2, w3t, b3):
    grid = (B // BLK,)
    out = pl.pallas_call(
        _gating_body,
        grid=grid,
        in_specs=[
            pl.BlockSpec((BLK, D), lambda i: (i, 0)),
            pl.BlockSpec((D, H1), lambda i: (0, 0)),
            pl.BlockSpec((1, H1), lambda i: (0, 0)),
            pl.BlockSpec((H1, H2), lambda i: (0, 0)),
            pl.BlockSpec((1, H2), lambda i: (0, 0)),
            pl.BlockSpec((H2, E), lambda i: (0, 0)),
            pl.BlockSpec((1, E), lambda i: (0, 0)),
        ],
        out_specs=[
            pl.BlockSpec((BLK, E), lambda i: (i, 0)),
            pl.BlockSpec((BLK, TOP_K), lambda i: (i, 0)),
            pl.BlockSpec((BLK, TOP_K), lambda i: (i, 0)),
        ],
        out_shape=[
            jax.ShapeDtypeStruct((B, E), jnp.float32),
            jax.ShapeDtypeStruct((B, TOP_K), jnp.int32),
            jax.ShapeDtypeStruct((B, TOP_K), jnp.float32),
        ],
    )(x, w1t, b1, w2t, b2, w3t, b3)
    return out


def kernel(x, W1, b1, W2, b2, W3, b3):
    gate_scores, top_k_indices, top_k_scores = _gating(
        x,
        W1.T, b1.reshape(1, H1),
        W2.T, b2.reshape(1, H2),
        W3.T, b3.reshape(1, E),
    )
    return (gate_scores, top_k_indices, top_k_scores)
